# baseline (speedup 1.0000x reference)
import functools

import jax
import jax.numpy as jnp
from jax import lax
from jax.experimental import pallas as pl
from jax.experimental.pallas import tpu as pltpu
from jax.experimental.pallas import tpu_sc as plsc

_TEMP = 0.2
_THRES = 0.4
_NPI = 256
_D = 512
_NB = 32
_NLVL = 3


def _stage1_body(iou_ref, crop_ref, box_ref, out_ref):
    x = box_ref[...]
    z = crop_ref[0]
    nb2 = jnp.sum(x * x, axis=1, keepdims=True)
    inv_nb = jax.lax.rsqrt(jnp.maximum(nb2, 1e-24))
    nz2 = jnp.sum(z * z, axis=1, keepdims=True)
    inv_nz = jax.lax.rsqrt(jnp.maximum(nz2, 1e-24))
    zh = z * inv_nz
    dots = jax.lax.dot_general(
        x, zh, (((1,), (1,)), ((), ())),
        preferred_element_type=jnp.float32)
    cos = dots * inv_nb
    mask = (iou_ref[0] >= _THRES).astype(jnp.float32)
    sp = jnp.sum(cos * mask, axis=0, keepdims=True)
    sa = jnp.sum(cos, axis=0, keepdims=True)
    cp = jnp.sum(mask)
    lane = jax.lax.broadcasted_iota(jnp.int32, (1, 128), 1)
    row = jnp.zeros((1, 128), jnp.float32)
    for k in range(_NLVL):
        row = jnp.where(lane == 16 * k, sp[0, k], row)
        row = jnp.where(lane == 16 * (k + 3), sa[0, k], row)
    row = jnp.where(lane == 16 * 6, cp, row)
    out_ref[0] = row


def _stage2_body(stats_ref, binv_ref, out_ref):
    s = stats_ref[:, 0, :]
    cp = s[:, 96:97]
    cn = _NPI - cp
    lvl_tot = None
    for k in range(_NLVL):
        sp = s[:, 16 * k:16 * k + 1]
        sa = s[:, 16 * (k + 3):16 * (k + 3) + 1]
        sn = sa - sp
        sim_pos = -(sp / cp)
        sim_neg = -(sn / cn)
        pos = jnp.exp(sim_pos / _TEMP)
        neg = jnp.exp(sim_neg / _TEMP)
        lb = -jnp.log(pos / (pos + neg))
        lvl = jnp.sum(lb, axis=0, keepdims=True)
        lvl_tot = lvl if lvl_tot is None else jnp.minimum(lvl_tot, lvl)
    out_ref[...] = lvl_tot * binv_ref[0, 0]


def _stage1_tc(box, crop, iou3):
    return pl.pallas_call(
        _stage1_body,
        grid=(_NB,),
        in_specs=[
            pl.BlockSpec((1, _NPI, 1), lambda b: (b, 0, 0)),
            pl.BlockSpec((1, _NLVL, _D), lambda b: (b, 0, 0)),
            pl.BlockSpec((_NPI, _D), lambda b: (b, 0)),
        ],
        out_specs=pl.BlockSpec((1, 1, 128), lambda b: (b, 0, 0)),
        out_shape=jax.ShapeDtypeStruct((_NB, 1, 128), jnp.float32),
    )(iou3, crop, box)


def _stage2(stats, binv):
    return pl.pallas_call(
        _stage2_body,
        in_specs=[
            pl.BlockSpec((_NB, 1, 128), lambda: (0, 0, 0)),
            pl.BlockSpec(memory_space=pltpu.SMEM),
        ],
        out_specs=pl.BlockSpec((1, 1), lambda: (0, 0)),
        out_shape=jax.ShapeDtypeStruct((1, 1), jnp.float32),
    )(stats, binv)


_HALF = 128
_RUNROLL = 8
_NCH = _D // 16


def _rsqrt16(x):
    i = lax.bitcast_convert_type(x, jnp.int32)
    i = 0x5F3759DF - lax.shift_right_logical(i, 1)
    y = lax.bitcast_convert_type(i, jnp.float32)
    for _ in range(3):
        y = y * (1.5 - 0.5 * x * y * y)
    return y


def _bsum16(v):
    return jnp.full((16,), jnp.sum(v), jnp.float32)


def _sc_stage1_body(box_hbm, crop_hbm, iou_hbm, out_hbm,
                    xbuf, zbuf, ioubuf, statbuf, sem):
    wid = lax.axis_index("s") * 2 + lax.axis_index("c")
    base_row = wid * _NPI
    pltpu.sync_copy(iou_hbm.at[wid], ioubuf.at[pl.ds(0, _NPI)])
    pltpu.sync_copy(crop_hbm.at[wid], zbuf)
    for li in range(_NLVL):
        acc = jnp.zeros((16,), jnp.float32)
        for c in range(_NCH):
            zv = zbuf[li, pl.ds(16 * c, 16)]
            acc = acc + zv * zv
        inv_nz = _rsqrt16(jnp.maximum(_bsum16(acc), 1e-24))
        for c in range(_NCH):
            zbuf[li, pl.ds(16 * c, 16)] = zbuf[li, pl.ds(16 * c, 16)] * inv_nz

    zero = jnp.zeros((16,), jnp.float32)
    stats = (zero,) * 7

    for h in range(2):
        pltpu.sync_copy(box_hbm.at[pl.ds(base_row + h * _HALF, _HALF)], xbuf)

        def group_body(g, carry, h=h):
            dp0, dp1, dp2, da0, da1, da2, cp = carry
            r0 = g * _RUNROLL
            accs = []
            for j in range(_RUNROLL):
                accs.append([zero, zero, zero, zero])
            for c in range(_NCH):
                z0 = zbuf[0, pl.ds(16 * c, 16)]
                z1 = zbuf[1, pl.ds(16 * c, 16)]
                z2 = zbuf[2, pl.ds(16 * c, 16)]
                for j in range(_RUNROLL):
                    x = xbuf[r0 + j, pl.ds(16 * c, 16)]
                    a = accs[j]
                    a[0] = a[0] + x * x
                    a[1] = a[1] + x * z0
                    a[2] = a[2] + x * z1
                    a[3] = a[3] + x * z2
            iouv = ioubuf[pl.ds(h * _HALF + r0, 16)]
            for j in range(_RUNROLL):
                nb2, d0, d1, d2 = accs[j]
                inv_nb = _rsqrt16(jnp.maximum(_bsum16(nb2), 1e-24))
                cos0 = _bsum16(d0) * inv_nb
                cos1 = _bsum16(d1) * inv_nb
                cos2 = _bsum16(d2) * inv_nb
                iou = jnp.full((16,), iouv[j], jnp.float32)
                m = iou >= _THRES
                dp0 = dp0 + jnp.where(m, cos0, 0.0)
                dp1 = dp1 + jnp.where(m, cos1, 0.0)
                dp2 = dp2 + jnp.where(m, cos2, 0.0)
                da0 = da0 + cos0
                da1 = da1 + cos1
                da2 = da2 + cos2
                cp = cp + jnp.where(m, 1.0, 0.0)
            return (dp0, dp1, dp2, da0, da1, da2, cp)

        stats = lax.fori_loop(0, _HALF // _RUNROLL, group_body, stats)

    for k in range(7):
        statbuf[pl.ds(16 * k, 16)] = stats[k]
    statbuf[pl.ds(16 * 7, 16)] = zero
    pltpu.sync_copy(statbuf, out_hbm.at[wid])


def _stage1_sc(box, cropT, iou2):
    mesh = plsc.VectorSubcoreMesh(core_axis_name="c", subcore_axis_name="s")
    f = functools.partial(
        pl.kernel,
        out_type=jax.ShapeDtypeStruct((_NB, 128), jnp.float32),
        mesh=mesh,
        compiler_params=pltpu.CompilerParams(needs_layout_passes=False),
        scratch_types=[
            pltpu.VMEM((_HALF, _D), jnp.float32),
            pltpu.VMEM((_NLVL, _D), jnp.float32),
            pltpu.VMEM((_NPI + 16,), jnp.float32),
            pltpu.VMEM((128,), jnp.float32),
            pltpu.SemaphoreType.DMA,
        ],
    )(_sc_stage1_body)
    return f(box, cropT, iou2)


def kernel(box_cls_feat_con, crop_feat_con, batch_size, ious):
    cropT = jnp.transpose(crop_feat_con, (1, 0, 2))
    binv = (1.0 / jnp.asarray(batch_size, jnp.float32)).reshape(1, 1)
    iou2 = ious.reshape(_NB, _NPI)
    stats = _stage1_sc(box_cls_feat_con, cropT, iou2)
    loss = _stage2(stats.reshape(_NB, 1, 128), binv)
    return loss[0, 0]
